# Initial kernel scaffold; baseline (speedup 1.0000x reference)
import jax
import jax.numpy as jnp
from jax.experimental import pallas as pl


def kernel(feat_id, feat_text, edge_weight, W_id, W_text, W_rel, ln_gamma, ln_beta, edge_index, edge_type):
    raise NotImplementedError("write your pallas kernel here")


# speedup vs baseline: 10.4268x; 10.4268x over previous
import functools

import jax
import jax.numpy as jnp
from jax import lax
from jax.experimental import pallas as pl
from jax.experimental.pallas import tpu as pltpu
from jax.experimental.pallas import tpu_sc as plsc

_NC = 2
_NS = 16
_NW = _NC * _NS
_K = 80

_DN = (((1,), (1,)), ((), ()))


@functools.lru_cache(maxsize=None)
def _proj_kernel(N, D, R, BN):
    grid = N // BN

    def body(fid, ftx, wid, wtx, wrel, h0_ref, g_ref):
        i = pl.program_id(0)
        h0 = (lax.dot_general(fid[...], wid[...], _DN,
                              preferred_element_type=jnp.float32)
              + lax.dot_general(ftx[...], wtx[...], _DN,
                                preferred_element_type=jnp.float32)) * 0.5
        glob = lax.broadcasted_iota(jnp.int32, (BN, D), 0) + i * BN
        h0 = jnp.where(glob == 0, 0.0, h0)
        h0_ref[...] = h0
        for r in range(R):
            g_ref[r] = lax.dot_general(h0, wrel[r], _DN,
                                       preferred_element_type=jnp.float32)

    return pl.pallas_call(
        body,
        grid=(grid,),
        in_specs=[pl.BlockSpec((BN, D), lambda i: (i, 0)),
                  pl.BlockSpec((BN, D), lambda i: (i, 0)),
                  pl.BlockSpec((D, D), lambda i: (0, 0)),
                  pl.BlockSpec((D, D), lambda i: (0, 0)),
                  pl.BlockSpec((R, D, D), lambda i: (0, 0, 0))],
        out_specs=[pl.BlockSpec((BN, D), lambda i: (i, 0)),
                   pl.BlockSpec((R, BN, D), lambda i: (0, i, 0))],
        out_shape=[jax.ShapeDtypeStruct((N, D), jnp.float32),
                   jax.ShapeDtypeStruct((R, N, D), jnp.float32)],
    )


@functools.lru_cache(maxsize=None)
def _denom_kernel(N, E, RNpad):
    EPW = E // _NW
    CH = EPW // _K
    ZD = RNpad // _NS
    mesh = plsc.VectorSubcoreMesh(core_axis_name="c", subcore_axis_name="s")

    @functools.partial(
        pl.kernel,
        out_type=jax.ShapeDtypeStruct((_NC * RNpad,), jnp.float32),
        mesh=mesh,
        scratch_types=[
            pltpu.VMEM_SHARED((RNpad,), jnp.float32),
            pltpu.VMEM((ZD,), jnp.float32),
            pltpu.VMEM((EPW,), jnp.int32),
            pltpu.VMEM((EPW,), jnp.int32),
            pltpu.VMEM((EPW,), jnp.float32),
            pltpu.VMEM((_K,), jnp.int32),
        ],
    )
    def k(dst_h, typ_h, w_h, out_h, den_sh, zb, dstb, typb, wb, idxb):
        c = lax.axis_index("c")
        s = lax.axis_index("s")
        wid = c * _NS + s

        def zset(i, carry):
            zb[pl.ds(i * 16, 16)] = jnp.zeros((16,), jnp.float32)
            return carry

        lax.fori_loop(0, ZD // 16, zset, None)
        pltpu.sync_copy(zb, den_sh.at[pl.ds(s * ZD, ZD)])
        plsc.subcore_barrier()

        base_e = wid * EPW
        pltpu.sync_copy(dst_h.at[pl.ds(base_e, EPW)], dstb)
        pltpu.sync_copy(typ_h.at[pl.ds(base_e, EPW)], typb)
        pltpu.sync_copy(w_h.at[pl.ds(base_e, EPW)], wb)

        def chunk(g, carry):
            b = g * _K
            for j in range(_K // 16):
                t = typb[pl.ds(b + j * 16, 16)]
                d = dstb[pl.ds(b + j * 16, 16)]
                idxb[pl.ds(j * 16, 16)] = t * N + d
            pltpu.sync_copy(wb.at[pl.ds(b, _K)], den_sh.at[idxb], add=True)
            return carry

        lax.fori_loop(0, CH, chunk, None)
        plsc.subcore_barrier()
        pltpu.sync_copy(den_sh.at[pl.ds(s * ZD, ZD)], zb)
        pltpu.sync_copy(zb, out_h.at[pl.ds(c * RNpad + s * ZD, ZD)])

    return k


@functools.lru_cache(maxsize=None)
def _inv_kernel(RNpad):
    rows = RNpad // 128

    def body(p_ref, o_ref):
        tot = p_ref[0] + p_ref[1]
        o_ref[...] = 1.0 / jnp.maximum(tot, 1e-8)

    return pl.pallas_call(
        body,
        out_shape=jax.ShapeDtypeStruct((rows, 128), jnp.float32),
    )


@functools.lru_cache(maxsize=None)
def _message_kernel(N, E, D, RNpad, Npad):
    EPW = E // _NW
    SCH = 2000
    NSC = EPW // SCH
    CH = SCH // _K
    RPW = Npad // _NS
    NZC = RPW // _K
    mesh = plsc.VectorSubcoreMesh(core_axis_name="c", subcore_axis_name="s")

    @functools.partial(
        pl.kernel,
        out_type=jax.ShapeDtypeStruct((_NC * Npad, D), jnp.float32),
        mesh=mesh,
        compiler_params=pltpu.CompilerParams(needs_layout_passes=False),
        scratch_types=[
            pltpu.VMEM_SHARED((Npad, D), jnp.float32),
            pltpu.VMEM((SCH,), jnp.int32),
            pltpu.VMEM((SCH,), jnp.int32),
            pltpu.VMEM((SCH,), jnp.int32),
            pltpu.VMEM((SCH,), jnp.float32),
            pltpu.VMEM((_K,), jnp.int32),
            pltpu.VMEM((_K,), jnp.int32),
            pltpu.VMEM((_K,), jnp.int32),
            pltpu.VMEM((_K,), jnp.float32),
            pltpu.VMEM((_K,), jnp.float32),
            pltpu.VMEM((_K, D), jnp.float32),
            pltpu.SemaphoreType.DMA,
        ],
    )
    def k(g_h, src_h, dst_h, typ_h, w_h, inv_h, out_h,
          acc_sh, srcb, dstb, typb, wb, gidx, didx, iidx, ivb, wkb,
          rows_v, sem):
        c = lax.axis_index("c")
        s = lax.axis_index("s")
        wid = c * _NS + s

        nv = D // 16

        def zrow(i, carry):
            r = i // nv
            q = i % nv
            rows_v[r, pl.ds(q * 16, 16)] = jnp.zeros((16,), jnp.float32)
            return carry

        lax.fori_loop(0, _K * nv, zrow, None)
        for kk in range(NZC):
            pltpu.sync_copy(rows_v,
                            acc_sh.at[pl.ds(s * RPW + kk * _K, _K), :])
        plsc.subcore_barrier()

        def super_chunk(sc, carry):
            base_e = wid * EPW + sc * SCH
            pltpu.sync_copy(src_h.at[pl.ds(base_e, SCH)], srcb)
            pltpu.sync_copy(dst_h.at[pl.ds(base_e, SCH)], dstb)
            pltpu.sync_copy(typ_h.at[pl.ds(base_e, SCH)], typb)
            pltpu.sync_copy(w_h.at[pl.ds(base_e, SCH)], wb)

            def chunk(g, carry1):
                b = g * _K
                for j in range(_K // 16):
                    t = typb[pl.ds(b + j * 16, 16)]
                    sr = srcb[pl.ds(b + j * 16, 16)]
                    dd = dstb[pl.ds(b + j * 16, 16)]
                    gidx[pl.ds(j * 16, 16)] = t * N + sr
                    didx[pl.ds(j * 16, 16)] = dd
                    iidx[pl.ds(j * 16, 16)] = t * N + dd
                rows_cp = pltpu.async_copy(g_h.at[gidx], rows_v, sem)
                inv_cp = pltpu.async_copy(inv_h.at[iidx], ivb, sem)
                rows_cp.wait()
                inv_cp.wait()
                for j in range(_K // 16):
                    wkb[pl.ds(j * 16, 16)] = (
                        wb[pl.ds(b + j * 16, 16)] * ivb[pl.ds(j * 16, 16)])

                def scale(e, carry2):
                    wv = plsc.load_gather(
                        wkb, [jnp.full((16,), e, jnp.int32)])
                    for j in range(nv):
                        rows_v[e, pl.ds(j * 16, 16)] = (
                            rows_v[e, pl.ds(j * 16, 16)] * wv)
                    return carry2

                lax.fori_loop(0, _K, scale, None)
                pltpu.sync_copy(rows_v, acc_sh.at[didx], add=True)
                return carry1

            lax.fori_loop(0, CH, chunk, None)
            return carry

        lax.fori_loop(0, NSC, super_chunk, None)
        plsc.subcore_barrier()
        for kk in range(NZC):
            r0 = s * RPW + kk * _K
            pltpu.sync_copy(acc_sh.at[pl.ds(r0, _K), :], rows_v)
            pltpu.sync_copy(rows_v, out_h.at[pl.ds(c * Npad + r0, _K), :])

    return k


@functools.lru_cache(maxsize=None)
def _final_kernel(N, D, BN):
    grid = N // BN

    def body(h0, a0, a1, gm, bt, o_ref):
        i = pl.program_id(0)
        x = h0[...] + a0[...] + a1[...]
        mu = jnp.mean(x, axis=1, keepdims=True)
        xc = x - mu
        var = jnp.mean(xc * xc, axis=1, keepdims=True)
        y = xc * lax.rsqrt(var + 1e-5) * gm[...] + bt[...]
        glob = lax.broadcasted_iota(jnp.int32, (BN, D), 0) + i * BN
        o_ref[...] = jnp.where(glob == 0, 0.0, y)

    return pl.pallas_call(
        body,
        grid=(grid,),
        in_specs=[pl.BlockSpec((BN, D), lambda i: (i, 0)),
                  pl.BlockSpec((BN, D), lambda i: (i, 0)),
                  pl.BlockSpec((BN, D), lambda i: (i, 0)),
                  pl.BlockSpec((1, D), lambda i: (0, 0)),
                  pl.BlockSpec((1, D), lambda i: (0, 0))],
        out_specs=pl.BlockSpec((BN, D), lambda i: (i, 0)),
        out_shape=jax.ShapeDtypeStruct((N, D), jnp.float32),
    )


def kernel(feat_id, feat_text, edge_weight, W_id, W_text, W_rel,
           ln_gamma, ln_beta, edge_index, edge_type):
    N, D = feat_id.shape
    R = W_rel.shape[0]
    E = edge_type.shape[0]
    RNpad = ((R * N + 255) // 256) * 256

    src = edge_index[0].astype(jnp.int32)
    dst = edge_index[1].astype(jnp.int32)
    typ = edge_type.astype(jnp.int32)
    w = edge_weight.astype(jnp.float32)

    h0, G = _proj_kernel(N, D, R, 1000)(
        feat_id, feat_text, W_id, W_text, W_rel)
    Gf = G.reshape(R * N, D)

    den = _denom_kernel(N, E, RNpad)(dst, typ, w)
    inv = _inv_kernel(RNpad)(
        den.reshape(_NC, RNpad // 128, 128)).reshape(RNpad)

    Npad = ((N + 2047) // 2048) * 2048
    acc = _message_kernel(N, E, D, RNpad, Npad)(Gf, src, dst, typ, w, inv)

    out = _final_kernel(N, D, 1000)(
        h0, acc[:N], acc[Npad:Npad + N],
        ln_gamma.reshape(1, D), ln_beta.reshape(1, D))
    return out


# speedup vs baseline: 15.5592x; 1.4922x over previous
import functools

import jax
import jax.numpy as jnp
from jax import lax
from jax.experimental import pallas as pl
from jax.experimental.pallas import tpu as pltpu
from jax.experimental.pallas import tpu_sc as plsc

_NC = 2
_NS = 16
_NW = _NC * _NS
_K = 80

_DN = (((1,), (1,)), ((), ()))


@functools.lru_cache(maxsize=None)
def _proj_kernel(N, D, R, BN):
    grid = N // BN

    def body(fid, ftx, wid, wtx, wrel, h0_ref, g_ref):
        i = pl.program_id(0)
        h0 = (lax.dot_general(fid[...], wid[...], _DN,
                              preferred_element_type=jnp.float32)
              + lax.dot_general(ftx[...], wtx[...], _DN,
                                preferred_element_type=jnp.float32)) * 0.5
        glob = lax.broadcasted_iota(jnp.int32, (BN, D), 0) + i * BN
        h0 = jnp.where(glob == 0, 0.0, h0)
        h0_ref[...] = h0
        for r in range(R):
            g_ref[r] = lax.dot_general(h0, wrel[r], _DN,
                                       preferred_element_type=jnp.float32)

    return pl.pallas_call(
        body,
        grid=(grid,),
        in_specs=[pl.BlockSpec((BN, D), lambda i: (i, 0)),
                  pl.BlockSpec((BN, D), lambda i: (i, 0)),
                  pl.BlockSpec((D, D), lambda i: (0, 0)),
                  pl.BlockSpec((D, D), lambda i: (0, 0)),
                  pl.BlockSpec((R, D, D), lambda i: (0, 0, 0))],
        out_specs=[pl.BlockSpec((BN, D), lambda i: (i, 0)),
                   pl.BlockSpec((R, BN, D), lambda i: (0, i, 0))],
        out_shape=[jax.ShapeDtypeStruct((N, D), jnp.float32),
                   jax.ShapeDtypeStruct((R, N, D), jnp.float32)],
    )


@functools.lru_cache(maxsize=None)
def _denom_kernel(N, E, RNpad):
    EPW = E // _NW
    CH = EPW // _K
    ZD = RNpad // _NS
    mesh = plsc.VectorSubcoreMesh(core_axis_name="c", subcore_axis_name="s")

    @functools.partial(
        pl.kernel,
        out_type=jax.ShapeDtypeStruct((_NC * RNpad,), jnp.float32),
        mesh=mesh,
        scratch_types=[
            pltpu.VMEM_SHARED((RNpad,), jnp.float32),
            pltpu.VMEM((ZD,), jnp.float32),
            pltpu.VMEM((EPW,), jnp.int32),
            pltpu.VMEM((EPW,), jnp.int32),
            pltpu.VMEM((EPW,), jnp.float32),
            pltpu.VMEM((_K,), jnp.int32),
        ],
    )
    def k(dst_h, typ_h, w_h, out_h, den_sh, zb, dstb, typb, wb, idxb):
        c = lax.axis_index("c")
        s = lax.axis_index("s")
        wid = c * _NS + s

        def zset(i, carry):
            zb[pl.ds(i * 16, 16)] = jnp.zeros((16,), jnp.float32)
            return carry

        lax.fori_loop(0, ZD // 16, zset, None)
        pltpu.sync_copy(zb, den_sh.at[pl.ds(s * ZD, ZD)])
        plsc.subcore_barrier()

        base_e = wid * EPW
        pltpu.sync_copy(dst_h.at[pl.ds(base_e, EPW)], dstb)
        pltpu.sync_copy(typ_h.at[pl.ds(base_e, EPW)], typb)
        pltpu.sync_copy(w_h.at[pl.ds(base_e, EPW)], wb)

        def chunk(g, carry):
            b = g * _K
            for j in range(_K // 16):
                t = typb[pl.ds(b + j * 16, 16)]
                d = dstb[pl.ds(b + j * 16, 16)]
                idxb[pl.ds(j * 16, 16)] = t * N + d
            pltpu.sync_copy(wb.at[pl.ds(b, _K)], den_sh.at[idxb], add=True)
            return carry

        lax.fori_loop(0, CH, chunk, None)
        plsc.subcore_barrier()
        pltpu.sync_copy(den_sh.at[pl.ds(s * ZD, ZD)], zb)
        pltpu.sync_copy(zb, out_h.at[pl.ds(c * RNpad + s * ZD, ZD)])

    return k


@functools.lru_cache(maxsize=None)
def _inv_kernel(RNpad):
    rows = RNpad // 128

    def body(p_ref, o_ref):
        tot = p_ref[0] + p_ref[1]
        o_ref[...] = 1.0 / jnp.maximum(tot, 1e-8)

    return pl.pallas_call(
        body,
        out_shape=jax.ShapeDtypeStruct((rows, 128), jnp.float32),
    )


@functools.lru_cache(maxsize=None)
def _message_kernel(N, E, D, RNpad, Npad):
    EPW = E // _NW
    SCH = 2000
    NSC = EPW // SCH
    CH = SCH // _K
    RPW = Npad // _NS
    NZC = RPW // _K
    mesh = plsc.VectorSubcoreMesh(core_axis_name="c", subcore_axis_name="s")

    @functools.partial(
        pl.kernel,
        out_type=jax.ShapeDtypeStruct((_NC * Npad, D), jnp.float32),
        mesh=mesh,
        compiler_params=pltpu.CompilerParams(needs_layout_passes=False),
        scratch_types=[
            pltpu.VMEM_SHARED((Npad, D), jnp.float32),
            pltpu.VMEM((SCH,), jnp.int32),
            pltpu.VMEM((SCH,), jnp.int32),
            pltpu.VMEM((SCH,), jnp.int32),
            pltpu.VMEM((SCH,), jnp.float32),
            [pltpu.VMEM((_K,), jnp.int32)] * 2,
            [pltpu.VMEM((_K,), jnp.int32)] * 2,
            [pltpu.VMEM((_K,), jnp.int32)] * 2,
            [pltpu.VMEM((_K,), jnp.float32)] * 2,
            pltpu.VMEM((_K,), jnp.float32),
            [pltpu.VMEM((_K, D), jnp.float32)] * 2,
            [pltpu.SemaphoreType.DMA] * 2,
            [pltpu.SemaphoreType.DMA] * 2,
        ],
    )
    def k(g_h, src_h, dst_h, typ_h, w_h, inv_h, out_h,
          acc_sh, srcb, dstb, typb, wb, gidx, didx, iidx, ivb, wkb,
          rows_v, sem_r, sem_i):
        c = lax.axis_index("c")
        s = lax.axis_index("s")
        wid = c * _NS + s

        nv = D // 16

        def zrow(i, carry):
            r = i // nv
            q = i % nv
            rows_v[0][r, pl.ds(q * 16, 16)] = jnp.zeros((16,), jnp.float32)
            return carry

        lax.fori_loop(0, _K * nv, zrow, None)
        for kk in range(NZC):
            pltpu.sync_copy(rows_v[0],
                            acc_sh.at[pl.ds(s * RPW + kk * _K, _K), :])
        plsc.subcore_barrier()

        def build_fire(g, b):
            bofs = g * _K
            for j in range(_K // 16):
                t = typb[pl.ds(bofs + j * 16, 16)]
                sr = srcb[pl.ds(bofs + j * 16, 16)]
                dd = dstb[pl.ds(bofs + j * 16, 16)]
                gidx[b][pl.ds(j * 16, 16)] = t * N + sr
                didx[b][pl.ds(j * 16, 16)] = dd
                iidx[b][pl.ds(j * 16, 16)] = t * N + dd
            pltpu.async_copy(g_h.at[gidx[b]], rows_v[b], sem_r[b])
            pltpu.async_copy(inv_h.at[iidx[b]], ivb[b], sem_i[b])

        def drain(g, b):
            bofs = g * _K
            pltpu.make_async_copy(g_h.at[gidx[b]], rows_v[b],
                                  sem_r[b]).wait()
            pltpu.make_async_copy(inv_h.at[iidx[b]], ivb[b],
                                  sem_i[b]).wait()
            for j in range(_K // 16):
                wkb[pl.ds(j * 16, 16)] = (
                    wb[pl.ds(bofs + j * 16, 16)] * ivb[b][pl.ds(j * 16, 16)])

            def scale(e, carry2):
                wv = plsc.load_gather(wkb, [jnp.full((16,), e, jnp.int32)])
                for j in range(nv):
                    rows_v[b][e, pl.ds(j * 16, 16)] = (
                        rows_v[b][e, pl.ds(j * 16, 16)] * wv)
                return carry2

            lax.fori_loop(0, _K, scale, None)
            pltpu.sync_copy(rows_v[b], acc_sh.at[didx[b]], add=True)

        def super_chunk(sc, carry):
            base_e = wid * EPW + sc * SCH
            pltpu.sync_copy(src_h.at[pl.ds(base_e, SCH)], srcb)
            pltpu.sync_copy(dst_h.at[pl.ds(base_e, SCH)], dstb)
            pltpu.sync_copy(typ_h.at[pl.ds(base_e, SCH)], typb)
            pltpu.sync_copy(w_h.at[pl.ds(base_e, SCH)], wb)

            build_fire(0, 0)

            def pair(i, carry1):
                go = i * 2

                @pl.when(go + 1 < CH)
                def _():
                    build_fire(go + 1, 1)

                drain(go, 0)

                @pl.when(go + 1 < CH)
                def _():
                    @pl.when(go + 2 < CH)
                    def _():
                        build_fire(go + 2, 0)

                    drain(go + 1, 1)

                return carry1

            lax.fori_loop(0, (CH + 1) // 2, pair, None)
            return carry

        lax.fori_loop(0, NSC, super_chunk, None)
        plsc.subcore_barrier()
        for kk in range(NZC):
            r0 = s * RPW + kk * _K
            pltpu.sync_copy(acc_sh.at[pl.ds(r0, _K), :], rows_v[0])
            pltpu.sync_copy(rows_v[0], out_h.at[pl.ds(c * Npad + r0, _K), :])

    return k


@functools.lru_cache(maxsize=None)
def _final_kernel(N, D, BN):
    grid = N // BN

    def body(h0, a0, a1, gm, bt, o_ref):
        i = pl.program_id(0)
        x = h0[...] + a0[...] + a1[...]
        mu = jnp.mean(x, axis=1, keepdims=True)
        xc = x - mu
        var = jnp.mean(xc * xc, axis=1, keepdims=True)
        y = xc * lax.rsqrt(var + 1e-5) * gm[...] + bt[...]
        glob = lax.broadcasted_iota(jnp.int32, (BN, D), 0) + i * BN
        o_ref[...] = jnp.where(glob == 0, 0.0, y)

    return pl.pallas_call(
        body,
        grid=(grid,),
        in_specs=[pl.BlockSpec((BN, D), lambda i: (i, 0)),
                  pl.BlockSpec((BN, D), lambda i: (i, 0)),
                  pl.BlockSpec((BN, D), lambda i: (i, 0)),
                  pl.BlockSpec((1, D), lambda i: (0, 0)),
                  pl.BlockSpec((1, D), lambda i: (0, 0))],
        out_specs=pl.BlockSpec((BN, D), lambda i: (i, 0)),
        out_shape=jax.ShapeDtypeStruct((N, D), jnp.float32),
    )


def kernel(feat_id, feat_text, edge_weight, W_id, W_text, W_rel,
           ln_gamma, ln_beta, edge_index, edge_type):
    N, D = feat_id.shape
    R = W_rel.shape[0]
    E = edge_type.shape[0]
    RNpad = ((R * N + 255) // 256) * 256

    src = edge_index[0].astype(jnp.int32)
    dst = edge_index[1].astype(jnp.int32)
    typ = edge_type.astype(jnp.int32)
    w = edge_weight.astype(jnp.float32)

    h0, G = _proj_kernel(N, D, R, 1000)(
        feat_id, feat_text, W_id, W_text, W_rel)
    Gf = G.reshape(R * N, D)

    den = _denom_kernel(N, E, RNpad)(dst, typ, w)
    inv = _inv_kernel(RNpad)(
        den.reshape(_NC, RNpad // 128, 128)).reshape(RNpad)

    Npad = ((N + 2047) // 2048) * 2048
    acc = _message_kernel(N, E, D, RNpad, Npad)(Gf, src, dst, typ, w, inv)

    out = _final_kernel(N, D, 1000)(
        h0, acc[:N], acc[Npad:Npad + N],
        ln_gamma.reshape(1, D), ln_beta.reshape(1, D))
    return out


# speedup vs baseline: 15.7264x; 1.0107x over previous
import functools

import jax
import jax.numpy as jnp
from jax import lax
from jax.experimental import pallas as pl
from jax.experimental.pallas import tpu as pltpu
from jax.experimental.pallas import tpu_sc as plsc

_NC = 2
_NS = 16
_NW = _NC * _NS
_K = 80

_DN = (((1,), (1,)), ((), ()))


@functools.lru_cache(maxsize=None)
def _proj_kernel(N, D, R, BN):
    grid = N // BN

    def body(fid, ftx, wid, wtx, wrel, h0_ref, g_ref):
        i = pl.program_id(0)
        h0 = (lax.dot_general(fid[...], wid[...], _DN,
                              preferred_element_type=jnp.float32)
              + lax.dot_general(ftx[...], wtx[...], _DN,
                                preferred_element_type=jnp.float32)) * 0.5
        glob = lax.broadcasted_iota(jnp.int32, (BN, D), 0) + i * BN
        h0 = jnp.where(glob == 0, 0.0, h0)
        h0_ref[...] = h0
        for r in range(R):
            g_ref[r] = lax.dot_general(h0, wrel[r], _DN,
                                       preferred_element_type=jnp.float32)

    return pl.pallas_call(
        body,
        grid=(grid,),
        in_specs=[pl.BlockSpec((BN, D), lambda i: (i, 0)),
                  pl.BlockSpec((BN, D), lambda i: (i, 0)),
                  pl.BlockSpec((D, D), lambda i: (0, 0)),
                  pl.BlockSpec((D, D), lambda i: (0, 0)),
                  pl.BlockSpec((R, D, D), lambda i: (0, 0, 0))],
        out_specs=[pl.BlockSpec((BN, D), lambda i: (i, 0)),
                   pl.BlockSpec((R, BN, D), lambda i: (0, i, 0))],
        out_shape=[jax.ShapeDtypeStruct((N, D), jnp.float32),
                   jax.ShapeDtypeStruct((R, N, D), jnp.float32)],
    )


@functools.lru_cache(maxsize=None)
def _denom_kernel(N, E, RNpad):
    EPW = E // _NW
    CH = EPW // _K
    ZD = RNpad // _NS
    mesh = plsc.VectorSubcoreMesh(core_axis_name="c", subcore_axis_name="s")

    @functools.partial(
        pl.kernel,
        out_type=jax.ShapeDtypeStruct((_NC * RNpad,), jnp.float32),
        mesh=mesh,
        scratch_types=[
            pltpu.VMEM_SHARED((RNpad,), jnp.float32),
            pltpu.VMEM((ZD,), jnp.float32),
            pltpu.VMEM((EPW,), jnp.int32),
            pltpu.VMEM((EPW,), jnp.int32),
            pltpu.VMEM((EPW,), jnp.float32),
            pltpu.VMEM((_K,), jnp.int32),
        ],
    )
    def k(dst_h, typ_h, w_h, out_h, den_sh, zb, dstb, typb, wb, idxb):
        c = lax.axis_index("c")
        s = lax.axis_index("s")
        wid = c * _NS + s

        def zset(i, carry):
            zb[pl.ds(i * 16, 16)] = jnp.zeros((16,), jnp.float32)
            return carry

        lax.fori_loop(0, ZD // 16, zset, None)
        pltpu.sync_copy(zb, den_sh.at[pl.ds(s * ZD, ZD)])
        plsc.subcore_barrier()

        base_e = wid * EPW
        pltpu.sync_copy(dst_h.at[pl.ds(base_e, EPW)], dstb)
        pltpu.sync_copy(typ_h.at[pl.ds(base_e, EPW)], typb)
        pltpu.sync_copy(w_h.at[pl.ds(base_e, EPW)], wb)

        def chunk(g, carry):
            b = g * _K
            for j in range(_K // 16):
                t = typb[pl.ds(b + j * 16, 16)]
                d = dstb[pl.ds(b + j * 16, 16)]
                idxb[pl.ds(j * 16, 16)] = t * N + d
            pltpu.sync_copy(wb.at[pl.ds(b, _K)], den_sh.at[idxb], add=True)
            return carry

        lax.fori_loop(0, CH, chunk, None)
        plsc.subcore_barrier()
        pltpu.sync_copy(den_sh.at[pl.ds(s * ZD, ZD)], zb)
        pltpu.sync_copy(zb, out_h.at[pl.ds(c * RNpad + s * ZD, ZD)])

    return k


@functools.lru_cache(maxsize=None)
def _inv_kernel(RNpad):
    rows = RNpad // 128

    def body(p_ref, o_ref):
        tot = p_ref[0] + p_ref[1]
        o_ref[...] = 1.0 / jnp.maximum(tot, 1e-8)

    return pl.pallas_call(
        body,
        out_shape=jax.ShapeDtypeStruct((rows, 128), jnp.float32),
    )


@functools.lru_cache(maxsize=None)
def _message_kernel(N, E, D, RNpad, Npad):
    EPW = E // _NW
    SCH = 2000
    NSC = EPW // SCH
    CH = SCH // _K
    RPW = Npad // _NS
    NZC = RPW // _K
    mesh = plsc.VectorSubcoreMesh(core_axis_name="c", subcore_axis_name="s")

    @functools.partial(
        pl.kernel,
        out_type=jax.ShapeDtypeStruct((_NC * Npad, D), jnp.float32),
        mesh=mesh,
        compiler_params=pltpu.CompilerParams(needs_layout_passes=False),
        scratch_types=[
            pltpu.VMEM_SHARED((Npad, D), jnp.float32),
            pltpu.VMEM((SCH,), jnp.int32),
            pltpu.VMEM((SCH,), jnp.int32),
            pltpu.VMEM((SCH,), jnp.int32),
            pltpu.VMEM((SCH,), jnp.float32),
            [pltpu.VMEM((_K,), jnp.int32)] * 2,
            [pltpu.VMEM((_K,), jnp.int32)] * 2,
            [pltpu.VMEM((_K,), jnp.int32)] * 2,
            [pltpu.VMEM((_K,), jnp.float32)] * 2,
            pltpu.VMEM((_K,), jnp.float32),
            [pltpu.VMEM((_K, D), jnp.float32)] * 2,
            [pltpu.SemaphoreType.DMA] * 2,
            [pltpu.SemaphoreType.DMA] * 2,
            [pltpu.SemaphoreType.DMA] * 2,
        ],
    )
    def k(g_h, src_h, dst_h, typ_h, w_h, inv_h, out_h,
          acc_sh, srcb, dstb, typb, wb, gidx, didx, iidx, ivb, wkb,
          rows_v, sem_r, sem_i, sem_s):
        c = lax.axis_index("c")
        s = lax.axis_index("s")
        wid = c * _NS + s

        nv = D // 16

        def zrow(i, carry):
            r = i // nv
            q = i % nv
            rows_v[0][r, pl.ds(q * 16, 16)] = jnp.zeros((16,), jnp.float32)
            return carry

        lax.fori_loop(0, _K * nv, zrow, None)
        for kk in range(NZC):
            pltpu.sync_copy(rows_v[0],
                            acc_sh.at[pl.ds(s * RPW + kk * _K, _K), :])
        plsc.subcore_barrier()

        for b in (0, 1):
            for j in range(_K // 16):
                didx[b][pl.ds(j * 16, 16)] = jnp.full((16,), N, jnp.int32)
            pltpu.async_copy(rows_v[b], acc_sh.at[didx[b]], sem_s[b],
                             add=True)

        def build_fire(g, b):
            pltpu.make_async_copy(rows_v[b], acc_sh.at[didx[b]],
                                  sem_s[b]).wait()
            bofs = g * _K
            for j in range(_K // 16):
                t = typb[pl.ds(bofs + j * 16, 16)]
                sr = srcb[pl.ds(bofs + j * 16, 16)]
                dd = dstb[pl.ds(bofs + j * 16, 16)]
                gidx[b][pl.ds(j * 16, 16)] = t * N + sr
                didx[b][pl.ds(j * 16, 16)] = dd
                iidx[b][pl.ds(j * 16, 16)] = t * N + dd
            pltpu.async_copy(g_h.at[gidx[b]], rows_v[b], sem_r[b])
            pltpu.async_copy(inv_h.at[iidx[b]], ivb[b], sem_i[b])

        def drain(g, b):
            bofs = g * _K
            pltpu.make_async_copy(g_h.at[gidx[b]], rows_v[b],
                                  sem_r[b]).wait()
            pltpu.make_async_copy(inv_h.at[iidx[b]], ivb[b],
                                  sem_i[b]).wait()
            for j in range(_K // 16):
                wkb[pl.ds(j * 16, 16)] = (
                    wb[pl.ds(bofs + j * 16, 16)] * ivb[b][pl.ds(j * 16, 16)])

            def scale(e, carry2):
                wv = plsc.load_gather(wkb, [jnp.full((16,), e, jnp.int32)])
                for j in range(nv):
                    rows_v[b][e, pl.ds(j * 16, 16)] = (
                        rows_v[b][e, pl.ds(j * 16, 16)] * wv)
                return carry2

            lax.fori_loop(0, _K, scale, None)
            pltpu.async_copy(rows_v[b], acc_sh.at[didx[b]], sem_s[b],
                             add=True)

        def super_chunk(sc, carry):
            base_e = wid * EPW + sc * SCH
            pltpu.sync_copy(src_h.at[pl.ds(base_e, SCH)], srcb)
            pltpu.sync_copy(dst_h.at[pl.ds(base_e, SCH)], dstb)
            pltpu.sync_copy(typ_h.at[pl.ds(base_e, SCH)], typb)
            pltpu.sync_copy(w_h.at[pl.ds(base_e, SCH)], wb)

            build_fire(0, 0)

            def pair(i, carry1):
                go = i * 2

                @pl.when(go + 1 < CH)
                def _():
                    build_fire(go + 1, 1)

                drain(go, 0)

                @pl.when(go + 1 < CH)
                def _():
                    @pl.when(go + 2 < CH)
                    def _():
                        build_fire(go + 2, 0)

                    drain(go + 1, 1)

                return carry1

            lax.fori_loop(0, (CH + 1) // 2, pair, None)
            return carry

        lax.fori_loop(0, NSC, super_chunk, None)
        for b in (0, 1):
            pltpu.make_async_copy(rows_v[b], acc_sh.at[didx[b]],
                                  sem_s[b]).wait()
        plsc.subcore_barrier()
        for kk in range(NZC):
            r0 = s * RPW + kk * _K
            pltpu.sync_copy(acc_sh.at[pl.ds(r0, _K), :], rows_v[0])
            pltpu.sync_copy(rows_v[0], out_h.at[pl.ds(c * Npad + r0, _K), :])

    return k


@functools.lru_cache(maxsize=None)
def _final_kernel(N, D, BN):
    grid = N // BN

    def body(h0, a0, a1, gm, bt, o_ref):
        i = pl.program_id(0)
        x = h0[...] + a0[...] + a1[...]
        mu = jnp.mean(x, axis=1, keepdims=True)
        xc = x - mu
        var = jnp.mean(xc * xc, axis=1, keepdims=True)
        y = xc * lax.rsqrt(var + 1e-5) * gm[...] + bt[...]
        glob = lax.broadcasted_iota(jnp.int32, (BN, D), 0) + i * BN
        o_ref[...] = jnp.where(glob == 0, 0.0, y)

    return pl.pallas_call(
        body,
        grid=(grid,),
        in_specs=[pl.BlockSpec((BN, D), lambda i: (i, 0)),
                  pl.BlockSpec((BN, D), lambda i: (i, 0)),
                  pl.BlockSpec((BN, D), lambda i: (i, 0)),
                  pl.BlockSpec((1, D), lambda i: (0, 0)),
                  pl.BlockSpec((1, D), lambda i: (0, 0))],
        out_specs=pl.BlockSpec((BN, D), lambda i: (i, 0)),
        out_shape=jax.ShapeDtypeStruct((N, D), jnp.float32),
    )


def kernel(feat_id, feat_text, edge_weight, W_id, W_text, W_rel,
           ln_gamma, ln_beta, edge_index, edge_type):
    N, D = feat_id.shape
    R = W_rel.shape[0]
    E = edge_type.shape[0]
    RNpad = ((R * N + 255) // 256) * 256

    src = edge_index[0].astype(jnp.int32)
    dst = edge_index[1].astype(jnp.int32)
    typ = edge_type.astype(jnp.int32)
    w = edge_weight.astype(jnp.float32)

    h0, G = _proj_kernel(N, D, R, 1000)(
        feat_id, feat_text, W_id, W_text, W_rel)
    Gf = G.reshape(R * N, D)

    den = _denom_kernel(N, E, RNpad)(dst, typ, w)
    inv = _inv_kernel(RNpad)(
        den.reshape(_NC, RNpad // 128, 128)).reshape(RNpad)

    Npad = ((N + 2047) // 2048) * 2048
    acc = _message_kernel(N, E, D, RNpad, Npad)(Gf, src, dst, typ, w, inv)

    out = _final_kernel(N, D, 1000)(
        h0, acc[:N], acc[Npad:Npad + N],
        ln_gamma.reshape(1, D), ln_beta.reshape(1, D))
    return out


# speedup vs baseline: 20.1650x; 1.2822x over previous
import functools

import jax
import jax.numpy as jnp
from jax import lax
from jax.experimental import pallas as pl
from jax.experimental.pallas import tpu as pltpu
from jax.experimental.pallas import tpu_sc as plsc

_NC = 2
_NS = 16
_NW = _NC * _NS
_K = 80

_DN = (((1,), (1,)), ((), ()))


@functools.lru_cache(maxsize=None)
def _proj_kernel(N, D, R, BN):
    grid = N // BN

    def body(fid, ftx, wid, wtx, wrel, h0_ref, g_ref):
        i = pl.program_id(0)
        h0 = (lax.dot_general(fid[...], wid[...], _DN,
                              preferred_element_type=jnp.float32)
              + lax.dot_general(ftx[...], wtx[...], _DN,
                                preferred_element_type=jnp.float32)) * 0.5
        glob = lax.broadcasted_iota(jnp.int32, (BN, D), 0) + i * BN
        h0 = jnp.where(glob == 0, 0.0, h0)
        h0_ref[...] = h0
        for r in range(R):
            g_ref[r] = lax.dot_general(h0, wrel[r], _DN,
                                       preferred_element_type=jnp.float32)

    return pl.pallas_call(
        body,
        grid=(grid,),
        in_specs=[pl.BlockSpec((BN, D), lambda i: (i, 0)),
                  pl.BlockSpec((BN, D), lambda i: (i, 0)),
                  pl.BlockSpec((D, D), lambda i: (0, 0)),
                  pl.BlockSpec((D, D), lambda i: (0, 0)),
                  pl.BlockSpec((R, D, D), lambda i: (0, 0, 0))],
        out_specs=[pl.BlockSpec((BN, D), lambda i: (i, 0)),
                   pl.BlockSpec((R, BN, D), lambda i: (0, i, 0))],
        out_shape=[jax.ShapeDtypeStruct((N, D), jnp.float32),
                   jax.ShapeDtypeStruct((R, N, D), jnp.float32)],
    )


@functools.lru_cache(maxsize=None)
def _denom_kernel(N, E, RNpad):
    EPW = E // _NW
    CH = EPW // _K
    ZD = RNpad // _NS
    mesh = plsc.VectorSubcoreMesh(core_axis_name="c", subcore_axis_name="s")

    @functools.partial(
        pl.kernel,
        out_type=jax.ShapeDtypeStruct((_NC * RNpad,), jnp.float32),
        mesh=mesh,
        scratch_types=[
            pltpu.VMEM_SHARED((RNpad,), jnp.float32),
            pltpu.VMEM((ZD,), jnp.float32),
            pltpu.VMEM((EPW,), jnp.int32),
            pltpu.VMEM((EPW,), jnp.int32),
            pltpu.VMEM((EPW,), jnp.float32),
            pltpu.VMEM((_K,), jnp.int32),
        ],
    )
    def k(dst_h, typ_h, w_h, out_h, den_sh, zb, dstb, typb, wb, idxb):
        c = lax.axis_index("c")
        s = lax.axis_index("s")
        wid = c * _NS + s

        def zset(i, carry):
            zb[pl.ds(i * 16, 16)] = jnp.zeros((16,), jnp.float32)
            return carry

        lax.fori_loop(0, ZD // 16, zset, None)
        pltpu.sync_copy(zb, den_sh.at[pl.ds(s * ZD, ZD)])
        plsc.subcore_barrier()

        base_e = wid * EPW
        pltpu.sync_copy(dst_h.at[pl.ds(base_e, EPW)], dstb)
        pltpu.sync_copy(typ_h.at[pl.ds(base_e, EPW)], typb)
        pltpu.sync_copy(w_h.at[pl.ds(base_e, EPW)], wb)

        def chunk(g, carry):
            b = g * _K
            for j in range(_K // 16):
                t = typb[pl.ds(b + j * 16, 16)]
                d = dstb[pl.ds(b + j * 16, 16)]
                idxb[pl.ds(j * 16, 16)] = t * N + d
            pltpu.sync_copy(wb.at[pl.ds(b, _K)], den_sh.at[idxb], add=True)
            return carry

        lax.fori_loop(0, CH, chunk, None)
        plsc.subcore_barrier()
        pltpu.sync_copy(den_sh.at[pl.ds(s * ZD, ZD)], zb)
        pltpu.sync_copy(zb, out_h.at[pl.ds(c * RNpad + s * ZD, ZD)])

    return k


@functools.lru_cache(maxsize=None)
def _inv_kernel(RNpad):
    rows = RNpad // 128

    def body(p_ref, o_ref):
        tot = p_ref[0] + p_ref[1]
        o_ref[...] = 1.0 / jnp.maximum(tot, 1e-8)

    return pl.pallas_call(
        body,
        out_shape=jax.ShapeDtypeStruct((rows, 128), jnp.float32),
    )


@functools.lru_cache(maxsize=None)
def _message_kernel(N, E, D, RNpad, Npad):
    EPW = E // _NW
    SCH = 2000
    NSC = EPW // SCH
    CH = SCH // _K
    RPW = Npad // _NS
    NZC = RPW // _K
    mesh = plsc.VectorSubcoreMesh(core_axis_name="c", subcore_axis_name="s")

    @functools.partial(
        pl.kernel,
        out_type=jax.ShapeDtypeStruct((_NC * Npad, D), jnp.float32),
        mesh=mesh,
        compiler_params=pltpu.CompilerParams(needs_layout_passes=False),
        scratch_types=[
            pltpu.VMEM_SHARED((Npad, D), jnp.float32),
            pltpu.VMEM((SCH,), jnp.int32),
            pltpu.VMEM((SCH,), jnp.int32),
            pltpu.VMEM((SCH,), jnp.int32),
            pltpu.VMEM((SCH,), jnp.float32),
            [pltpu.VMEM((_K,), jnp.int32)] * 2,
            [pltpu.VMEM((_K,), jnp.int32)] * 2,
            [pltpu.VMEM((_K,), jnp.int32)] * 2,
            [pltpu.VMEM((_K,), jnp.float32)] * 2,
            pltpu.VMEM((_K,), jnp.float32),
            [pltpu.VMEM((_K, D), jnp.float32)] * 2,
            [pltpu.SemaphoreType.DMA] * 2,
            [pltpu.SemaphoreType.DMA] * 2,
            [pltpu.SemaphoreType.DMA] * 2,
        ],
    )
    def k(g_h, src_h, dst_h, typ_h, w_h, inv_h, out_h,
          acc_sh, srcb, dstb, typb, wb, gidx, didx, iidx, ivb, wkb,
          rows_v, sem_r, sem_i, sem_s):
        c = lax.axis_index("c")
        s = lax.axis_index("s")
        wid = c * _NS + s

        nv = D // 16

        def zrow(i, carry):
            r = i // nv
            q = i % nv
            rows_v[0][r, pl.ds(q * 16, 16)] = jnp.zeros((16,), jnp.float32)
            return carry

        lax.fori_loop(0, _K * nv, zrow, None)
        for kk in range(NZC):
            pltpu.sync_copy(rows_v[0],
                            acc_sh.at[pl.ds(s * RPW + kk * _K, _K), :])
        plsc.subcore_barrier()

        for b in (0, 1):
            for j in range(_K // 16):
                didx[b][pl.ds(j * 16, 16)] = jnp.full((16,), N, jnp.int32)
            pltpu.async_copy(rows_v[b], acc_sh.at[didx[b]], sem_s[b],
                             add=True)

        def build_fire(g, b):
            pltpu.make_async_copy(rows_v[b], acc_sh.at[didx[b]],
                                  sem_s[b]).wait()
            bofs = g * _K
            for j in range(_K // 16):
                t = typb[pl.ds(bofs + j * 16, 16)]
                sr = srcb[pl.ds(bofs + j * 16, 16)]
                dd = dstb[pl.ds(bofs + j * 16, 16)]
                gidx[b][pl.ds(j * 16, 16)] = t * N + sr
                didx[b][pl.ds(j * 16, 16)] = dd
                iidx[b][pl.ds(j * 16, 16)] = t * N + dd
            pltpu.async_copy(g_h.at[gidx[b]], rows_v[b], sem_r[b])
            pltpu.async_copy(inv_h.at[iidx[b]], ivb[b], sem_i[b])

        def drain(g, b):
            bofs = g * _K
            pltpu.make_async_copy(g_h.at[gidx[b]], rows_v[b],
                                  sem_r[b]).wait()
            pltpu.make_async_copy(inv_h.at[iidx[b]], ivb[b],
                                  sem_i[b]).wait()
            for j in range(_K // 16):
                wkb[pl.ds(j * 16, 16)] = (
                    wb[pl.ds(bofs + j * 16, 16)] * ivb[b][pl.ds(j * 16, 16)])

            @functools.partial(plsc.parallel_loop, 0, _K, unroll=4)
            def scale(e):
                wv = plsc.load_gather(wkb, [jnp.full((16,), e, jnp.int32)])
                for j in range(nv):
                    rows_v[b][e, pl.ds(j * 16, 16)] = (
                        rows_v[b][e, pl.ds(j * 16, 16)] * wv)
            pltpu.async_copy(rows_v[b], acc_sh.at[didx[b]], sem_s[b],
                             add=True)

        def super_chunk(sc, carry):
            base_e = wid * EPW + sc * SCH
            pltpu.sync_copy(src_h.at[pl.ds(base_e, SCH)], srcb)
            pltpu.sync_copy(dst_h.at[pl.ds(base_e, SCH)], dstb)
            pltpu.sync_copy(typ_h.at[pl.ds(base_e, SCH)], typb)
            pltpu.sync_copy(w_h.at[pl.ds(base_e, SCH)], wb)

            build_fire(0, 0)

            def pair(i, carry1):
                go = i * 2

                @pl.when(go + 1 < CH)
                def _():
                    build_fire(go + 1, 1)

                drain(go, 0)

                @pl.when(go + 1 < CH)
                def _():
                    @pl.when(go + 2 < CH)
                    def _():
                        build_fire(go + 2, 0)

                    drain(go + 1, 1)

                return carry1

            lax.fori_loop(0, (CH + 1) // 2, pair, None)
            return carry

        lax.fori_loop(0, NSC, super_chunk, None)
        for b in (0, 1):
            pltpu.make_async_copy(rows_v[b], acc_sh.at[didx[b]],
                                  sem_s[b]).wait()
        plsc.subcore_barrier()
        for kk in range(NZC):
            r0 = s * RPW + kk * _K
            pltpu.sync_copy(acc_sh.at[pl.ds(r0, _K), :], rows_v[0])
            pltpu.sync_copy(rows_v[0], out_h.at[pl.ds(c * Npad + r0, _K), :])

    return k


@functools.lru_cache(maxsize=None)
def _final_kernel(N, D, BN):
    grid = N // BN

    def body(h0, a0, a1, gm, bt, o_ref):
        i = pl.program_id(0)
        x = h0[...] + a0[...] + a1[...]
        mu = jnp.mean(x, axis=1, keepdims=True)
        xc = x - mu
        var = jnp.mean(xc * xc, axis=1, keepdims=True)
        y = xc * lax.rsqrt(var + 1e-5) * gm[...] + bt[...]
        glob = lax.broadcasted_iota(jnp.int32, (BN, D), 0) + i * BN
        o_ref[...] = jnp.where(glob == 0, 0.0, y)

    return pl.pallas_call(
        body,
        grid=(grid,),
        in_specs=[pl.BlockSpec((BN, D), lambda i: (i, 0)),
                  pl.BlockSpec((BN, D), lambda i: (i, 0)),
                  pl.BlockSpec((BN, D), lambda i: (i, 0)),
                  pl.BlockSpec((1, D), lambda i: (0, 0)),
                  pl.BlockSpec((1, D), lambda i: (0, 0))],
        out_specs=pl.BlockSpec((BN, D), lambda i: (i, 0)),
        out_shape=jax.ShapeDtypeStruct((N, D), jnp.float32),
    )


def kernel(feat_id, feat_text, edge_weight, W_id, W_text, W_rel,
           ln_gamma, ln_beta, edge_index, edge_type):
    N, D = feat_id.shape
    R = W_rel.shape[0]
    E = edge_type.shape[0]
    RNpad = ((R * N + 255) // 256) * 256

    src = edge_index[0].astype(jnp.int32)
    dst = edge_index[1].astype(jnp.int32)
    typ = edge_type.astype(jnp.int32)
    w = edge_weight.astype(jnp.float32)

    h0, G = _proj_kernel(N, D, R, 1000)(
        feat_id, feat_text, W_id, W_text, W_rel)
    Gf = G.reshape(R * N, D)

    den = _denom_kernel(N, E, RNpad)(dst, typ, w)
    inv = _inv_kernel(RNpad)(
        den.reshape(_NC, RNpad // 128, 128)).reshape(RNpad)

    Npad = ((N + 2047) // 2048) * 2048
    acc = _message_kernel(N, E, D, RNpad, Npad)(Gf, src, dst, typ, w, inv)

    out = _final_kernel(N, D, 1000)(
        h0, acc[:N], acc[Npad:Npad + N],
        ln_gamma.reshape(1, D), ln_beta.reshape(1, D))
    return out
